# baseline (speedup 1.0000x reference)
import jax
import jax.numpy as jnp
from jax import lax
from jax.experimental import pallas as pl
from jax.experimental.pallas import tpu as pltpu
from jax.experimental.pallas import tpu_sc as plsc

N_ATOM_BASIS = 128
VOCAB = 100
NUM_ATOMS = 100000

_BLK = 128
_NUM_BLOCKS = 782
_LAST_START = NUM_ATOMS - _BLK
_NW = 32
_ITERS = 25


def _gather_kernel(table_hbm, idx_hbm, out_hbm, idx_v, rows_v, sem):
    wid = lax.axis_index("s") * 2 + lax.axis_index("c")

    def body(i, _):
        b = jnp.minimum(wid + i * _NW, _NUM_BLOCKS - 1)
        start = jnp.minimum(b * _BLK, _LAST_START)
        pltpu.sync_copy(idx_hbm.at[pl.ds(start, _BLK)], idx_v)
        pltpu.async_copy(table_hbm.at[idx_v], rows_v, sem).wait()
        pltpu.sync_copy(rows_v, out_hbm.at[pl.ds(start, _BLK)])
        return ()

    lax.fori_loop(0, _ITERS, body, ())


def _zeros_body(o_ref):
    o_ref[...] = jnp.zeros_like(o_ref)


@jax.jit
def kernel(z_number, atom_embed_weight):
    z = z_number.astype(jnp.int32)

    mesh = plsc.VectorSubcoreMesh(core_axis_name="c", subcore_axis_name="s")
    gather = pl.kernel(
        _gather_kernel,
        mesh=mesh,
        out_type=jax.ShapeDtypeStruct((NUM_ATOMS, N_ATOM_BASIS), jnp.float32),
        scratch_types=[
            pltpu.VMEM((_BLK,), jnp.int32),
            pltpu.VMEM((_BLK, N_ATOM_BASIS), jnp.float32),
            pltpu.SemaphoreType.DMA,
        ],
    )
    s_i = gather(atom_embed_weight, z)

    v_flat = pl.pallas_call(
        _zeros_body,
        out_shape=jax.ShapeDtypeStruct((NUM_ATOMS, 3 * N_ATOM_BASIS), jnp.float32),
        grid=(50,),
        out_specs=pl.BlockSpec((2000, 3 * N_ATOM_BASIS), lambda i: (i, 0)),
    )()
    v_i = v_flat.reshape(NUM_ATOMS, N_ATOM_BASIS, 3)
    return (s_i, v_i)


# speedup vs baseline: 3.0940x; 3.0940x over previous
import jax
import jax.numpy as jnp
from jax import lax
from jax.experimental import pallas as pl
from jax.experimental.pallas import tpu as pltpu
from jax.experimental.pallas import tpu_sc as plsc

N_ATOM_BASIS = 128
VOCAB = 100
NUM_ATOMS = 100000

_BLK = 128
_NUM_BLOCKS = 782
_LAST_START = NUM_ATOMS - _BLK
_NW = 32
_ITERS = 25


def _gather_kernel(table_hbm, idx_hbm, out_hbm, idx_v, rows_v, sem):
    wid = lax.axis_index("s") * 2 + lax.axis_index("c")

    def body(i, _):
        b = jnp.minimum(wid + i * _NW, _NUM_BLOCKS - 1)
        start = jnp.minimum(b * _BLK, _LAST_START)
        pltpu.sync_copy(idx_hbm.at[pl.ds(start, _BLK)], idx_v)
        pltpu.async_copy(table_hbm.at[idx_v], rows_v, sem).wait()
        pltpu.sync_copy(rows_v, out_hbm.at[pl.ds(start, _BLK)])
        return ()

    lax.fori_loop(0, _ITERS, body, ())


def _zeros_body(o_ref):
    o_ref[...] = jnp.zeros_like(o_ref)


@jax.jit
def kernel(z_number, atom_embed_weight):
    z = z_number.astype(jnp.int32)

    mesh = plsc.VectorSubcoreMesh(core_axis_name="c", subcore_axis_name="s")
    gather = pl.kernel(
        _gather_kernel,
        mesh=mesh,
        out_type=jax.ShapeDtypeStruct((NUM_ATOMS, N_ATOM_BASIS), jnp.float32),
        scratch_types=[
            pltpu.VMEM((_BLK,), jnp.int32),
            pltpu.VMEM((_BLK, N_ATOM_BASIS), jnp.float32),
            pltpu.SemaphoreType.DMA,
        ],
    )
    s_i = gather(atom_embed_weight, z)

    v_planes = pl.pallas_call(
        _zeros_body,
        out_shape=jax.ShapeDtypeStruct((3, NUM_ATOMS, N_ATOM_BASIS), jnp.float32),
        grid=(50,),
        out_specs=pl.BlockSpec((3, 2000, N_ATOM_BASIS), lambda i: (0, i, 0)),
    )()
    v_i = jnp.transpose(v_planes, (1, 2, 0))
    return (s_i, v_i)


# speedup vs baseline: 3.1641x; 1.0226x over previous
import jax
import jax.numpy as jnp
from jax import lax
from jax.experimental import pallas as pl
from jax.experimental.pallas import tpu as pltpu
from jax.experimental.pallas import tpu_sc as plsc

N_ATOM_BASIS = 128
VOCAB = 100
NUM_ATOMS = 100000

_BLK = 128
_NW = 32
_CHUNK = 3128
_LAST_CHUNK_START = NUM_ATOMS - _CHUNK
_ITERS = 25
_LAST_OFF = _CHUNK - _BLK
_NBUF = 6
_DEPTH = 3


def _gather_kernel(table_hbm, idx_hbm, out_hbm, idx_v, *bufs):
    rows = bufs[:_NBUF]
    gsem = bufs[_NBUF:2 * _NBUF]
    ssem = bufs[2 * _NBUF:3 * _NBUF]
    wid = lax.axis_index("s") * 2 + lax.axis_index("c")
    chunk = jnp.minimum(wid * _CHUNK, _LAST_CHUNK_START)

    pltpu.sync_copy(idx_hbm.at[pl.ds(chunk, _CHUNK)], idx_v)

    gathers, stores = [], []
    for step in range(_ITERS + _DEPTH):
        if step < _ITERS:
            i = step
            b = i % _NBUF
            off = min(i * _BLK, _LAST_OFF)
            if i >= _NBUF:
                stores[i - _NBUF].wait()
            gathers.append(
                pltpu.async_copy(
                    table_hbm.at[idx_v.at[pl.ds(off, _BLK)]], rows[b], gsem[b]
                )
            )
        if step >= _DEPTH:
            k = step - _DEPTH
            b = k % _NBUF
            off = min(k * _BLK, _LAST_OFF)
            gathers[k].wait()
            stores.append(
                pltpu.async_copy(
                    rows[b], out_hbm.at[pl.ds(chunk + off, _BLK)], ssem[b]
                )
            )
    for k in range(max(0, _ITERS - _NBUF), _ITERS):
        stores[k].wait()


def _zeros_body(o_ref):
    o_ref[...] = jnp.zeros_like(o_ref)


@jax.jit
def kernel(z_number, atom_embed_weight):
    z = z_number.astype(jnp.int32)

    mesh = plsc.VectorSubcoreMesh(core_axis_name="c", subcore_axis_name="s")
    gather = pl.kernel(
        _gather_kernel,
        mesh=mesh,
        out_type=jax.ShapeDtypeStruct((NUM_ATOMS, N_ATOM_BASIS), jnp.float32),
        scratch_types=(
            [pltpu.VMEM((_CHUNK,), jnp.int32)]
            + [pltpu.VMEM((_BLK, N_ATOM_BASIS), jnp.float32)] * _NBUF
            + [pltpu.SemaphoreType.DMA] * (2 * _NBUF)
        ),
    )
    s_i = gather(atom_embed_weight, z)

    v_planes = pl.pallas_call(
        _zeros_body,
        out_shape=jax.ShapeDtypeStruct((3, NUM_ATOMS, N_ATOM_BASIS), jnp.float32),
        grid=(50,),
        out_specs=pl.BlockSpec((3, 2000, N_ATOM_BASIS), lambda i: (0, i, 0)),
    )()
    v_i = jnp.transpose(v_planes, (1, 2, 0))
    return (s_i, v_i)


# speedup vs baseline: 8.3537x; 2.6402x over previous
import jax
import jax.numpy as jnp
from jax import lax
from jax.experimental import pallas as pl
from jax.experimental.pallas import tpu as pltpu
from jax.experimental.pallas import tpu_sc as plsc

N_ATOM_BASIS = 128
VOCAB = 100
NUM_ATOMS = 100000

_BLK = 128
_NW = 32
_CHUNK = 3128
_LAST_CHUNK_START = NUM_ATOMS - _CHUNK
_ITERS = 25
_LAST_OFF = _CHUNK - _BLK
_NBUF = 6
_DEPTH = 3


def _gather_kernel(table_hbm, idx_hbm, out_hbm, table_sh, idx_v, *bufs):
    rows = bufs[:_NBUF]
    gsem = bufs[_NBUF:2 * _NBUF]
    ssem = bufs[2 * _NBUF:3 * _NBUF]
    sid = lax.axis_index("s")
    wid = sid * 2 + lax.axis_index("c")
    chunk = jnp.minimum(wid * _CHUNK, _LAST_CHUNK_START)

    @pl.when(sid == 0)
    def _stage_table():
        pltpu.sync_copy(table_hbm, table_sh)

    pltpu.sync_copy(idx_hbm.at[pl.ds(chunk, _CHUNK)], idx_v)
    plsc.subcore_barrier()

    gathers, stores = [], []
    for step in range(_ITERS + _DEPTH):
        if step < _ITERS:
            i = step
            b = i % _NBUF
            off = min(i * _BLK, _LAST_OFF)
            if i >= _NBUF:
                stores[i - _NBUF].wait()
            gathers.append(
                pltpu.async_copy(
                    table_sh.at[idx_v.at[pl.ds(off, _BLK)]], rows[b], gsem[b]
                )
            )
        if step >= _DEPTH:
            k = step - _DEPTH
            b = k % _NBUF
            off = min(k * _BLK, _LAST_OFF)
            gathers[k].wait()
            stores.append(
                pltpu.async_copy(
                    rows[b], out_hbm.at[pl.ds(chunk + off, _BLK)], ssem[b]
                )
            )
    for k in range(max(0, _ITERS - _NBUF), _ITERS):
        stores[k].wait()


def _zeros_body(o_ref):
    o_ref[...] = jnp.zeros_like(o_ref)


@jax.jit
def kernel(z_number, atom_embed_weight):
    z = z_number.astype(jnp.int32)

    mesh = plsc.VectorSubcoreMesh(core_axis_name="c", subcore_axis_name="s")
    gather = pl.kernel(
        _gather_kernel,
        mesh=mesh,
        out_type=jax.ShapeDtypeStruct((NUM_ATOMS, N_ATOM_BASIS), jnp.float32),
        scratch_types=(
            [pltpu.VMEM_SHARED((VOCAB, N_ATOM_BASIS), jnp.float32)]
            + [pltpu.VMEM((_CHUNK,), jnp.int32)]
            + [pltpu.VMEM((_BLK, N_ATOM_BASIS), jnp.float32)] * _NBUF
            + [pltpu.SemaphoreType.DMA] * (2 * _NBUF)
        ),
    )
    s_i = gather(atom_embed_weight, z)

    v_planes = pl.pallas_call(
        _zeros_body,
        out_shape=jax.ShapeDtypeStruct((3, NUM_ATOMS, N_ATOM_BASIS), jnp.float32),
        grid=(50,),
        out_specs=pl.BlockSpec((3, 2000, N_ATOM_BASIS), lambda i: (0, i, 0)),
    )()
    v_i = jnp.transpose(v_planes, (1, 2, 0))
    return (s_i, v_i)


# speedup vs baseline: 8.3820x; 1.0034x over previous
import jax
import jax.numpy as jnp
from jax import lax
from jax.experimental import pallas as pl
from jax.experimental.pallas import tpu as pltpu
from jax.experimental.pallas import tpu_sc as plsc

N_ATOM_BASIS = 128
VOCAB = 100
NUM_ATOMS = 100000

_BLK = 128
_NW = 32
_CHUNK = 3128
_LAST_CHUNK_START = NUM_ATOMS - _CHUNK
_ITERS = 25
_LAST_OFF = _CHUNK - _BLK
_NBUF = 6
_DEPTH = 3


def _gather_kernel(table_hbm, idx_hbm, out_hbm, table_sh, idx_v, *bufs):
    rows = bufs[:_NBUF]
    gsem = bufs[_NBUF:2 * _NBUF]
    ssem = bufs[2 * _NBUF:3 * _NBUF]
    sid = lax.axis_index("s")
    wid = sid * 2 + lax.axis_index("c")
    chunk = jnp.minimum(wid * _CHUNK, _LAST_CHUNK_START)

    @pl.when(sid == 0)
    def _stage_table():
        pltpu.sync_copy(table_hbm, table_sh)

    pltpu.sync_copy(idx_hbm.at[pl.ds(chunk, _CHUNK)], idx_v)
    plsc.subcore_barrier()

    gathers, stores = [], []
    for step in range(_ITERS + _DEPTH):
        if step < _ITERS:
            i = step
            b = i % _NBUF
            off = min(i * _BLK, _LAST_OFF)
            if i >= _NBUF:
                stores[i - _NBUF].wait()
            gathers.append(
                pltpu.async_copy(
                    table_sh.at[idx_v.at[pl.ds(off, _BLK)]], rows[b], gsem[b]
                )
            )
        if step >= _DEPTH:
            k = step - _DEPTH
            b = k % _NBUF
            off = min(k * _BLK, _LAST_OFF)
            gathers[k].wait()
            stores.append(
                pltpu.async_copy(
                    rows[b], out_hbm.at[pl.ds(chunk + off, _BLK)], ssem[b]
                )
            )
    for k in range(max(0, _ITERS - _NBUF), _ITERS):
        stores[k].wait()


def _zeros_body(o_ref):
    o_ref[...] = jnp.zeros_like(o_ref)


@jax.jit
def kernel(z_number, atom_embed_weight):
    z = z_number.astype(jnp.int32)

    mesh = plsc.VectorSubcoreMesh(core_axis_name="c", subcore_axis_name="s")
    gather = pl.kernel(
        _gather_kernel,
        mesh=mesh,
        out_type=jax.ShapeDtypeStruct((NUM_ATOMS, N_ATOM_BASIS), jnp.float32),
        scratch_types=(
            [pltpu.VMEM_SHARED((VOCAB, N_ATOM_BASIS), jnp.float32)]
            + [pltpu.VMEM((_CHUNK,), jnp.int32)]
            + [pltpu.VMEM((_BLK, N_ATOM_BASIS), jnp.float32)] * _NBUF
            + [pltpu.SemaphoreType.DMA] * (2 * _NBUF)
        ),
    )
    s_i = gather(atom_embed_weight, z)

    v_planes = pl.pallas_call(
        _zeros_body,
        out_shape=jax.ShapeDtypeStruct((3, NUM_ATOMS, N_ATOM_BASIS), jnp.float32),
        grid=(20,),
        out_specs=pl.BlockSpec((3, 5000, N_ATOM_BASIS), lambda i: (0, i, 0)),
    )()
    v_i = jnp.transpose(v_planes, (1, 2, 0))
    return (s_i, v_i)
